# baseline (speedup 1.0000x reference)
import functools

import numpy as np
import jax
import jax.numpy as jnp
from jax import lax
from jax.experimental import pallas as pl
from jax.experimental.pallas import tpu as pltpu
from jax.experimental.pallas import tpu_sc as plsc

N_NODES = 10000
D = 128
DH = D // 2

NC = 2
NS = 16
NW = NC * NS
CH = 96
N_PAD = 10240
RPT = N_PAD // NS

_COLS_A = (np.arange(DH) // 16) * 32 + np.arange(DH) % 16
_COLS_B = _COLS_A + 16

_RPT_CHUNKS = [(t * CH, CH) for t in range(RPT // CH)]
if RPT % CH:
    _RPT_CHUNKS.append(((RPT // CH) * CH, RPT % CH))

_LOG2 = 0.6931471805599453


def _softplus(x):
    return jnp.logaddexp(x, 0.0)


def _pack_bf16_pairs(a, b):
    au = lax.bitcast_convert_type(a.astype(jnp.bfloat16), jnp.uint16)
    bu = lax.bitcast_convert_type(b.astype(jnp.bfloat16), jnp.uint16)
    return au.astype(jnp.uint32) | (bu.astype(jnp.uint32) << 16)


def _node_body(emb_ref, wiT_ref, bi_ref, wjT_ref, bj_ref, xi_ref, y_ref):
    x = _softplus(emb_ref[...]) - _LOG2
    xi_ref[...] = _softplus(
        jnp.dot(x, wiT_ref[...], preferred_element_type=jnp.float32) + bi_ref[...])
    y_ref[...] = _softplus(
        jnp.dot(x, wjT_ref[...], preferred_element_type=jnp.float32) + bj_ref[...])


def _node_stage(emb, wiT, bi2, wjT, bj2):
    nb = 1000
    return pl.pallas_call(
        _node_body,
        grid=(N_NODES // nb,),
        in_specs=[
            pl.BlockSpec((nb, D), lambda i: (i, 0)),
            pl.BlockSpec((D, D), lambda i: (0, 0)),
            pl.BlockSpec((1, D), lambda i: (0, 0)),
            pl.BlockSpec((D, D), lambda i: (0, 0)),
            pl.BlockSpec((1, D), lambda i: (0, 0)),
        ],
        out_specs=[pl.BlockSpec((nb, D), lambda i: (i, 0)),
                   pl.BlockSpec((nb, D), lambda i: (i, 0))],
        out_shape=[jax.ShapeDtypeStruct((N_NODES, D), jnp.float32),
                   jax.ShapeDtypeStruct((N_NODES, D), jnp.float32)],
    )(emb, wiT, bi2, wjT, bj2)


def _fp_body(f_ref, gTa_ref, gTb_ref, out_ref):
    fa = jnp.dot(f_ref[...], gTa_ref[...], preferred_element_type=jnp.float32)
    fb = jnp.dot(f_ref[...], gTb_ref[...], preferred_element_type=jnp.float32)
    out_ref[...] = _pack_bf16_pairs(fa, fb)


def _fp_stage(f_pad, gTa, gTb, e_pad):
    eb = 2048
    return pl.pallas_call(
        _fp_body,
        grid=(e_pad // eb,),
        in_specs=[pl.BlockSpec((eb, 16), lambda i: (i, 0)),
                  pl.BlockSpec((16, DH), lambda i: (0, 0)),
                  pl.BlockSpec((16, DH), lambda i: (0, 0))],
        out_specs=pl.BlockSpec((eb, DH), lambda i: (i, 0)),
        out_shape=jax.ShapeDtypeStruct((e_pad, DH), jnp.uint32),
    )(f_pad, gTa, gTb)


def _make_edge_stage(e_pad):
    epw = e_pad // NW
    nchunk = epw // CH

    mesh = plsc.VectorSubcoreMesh(core_axis_name="c", subcore_axis_name="s")

    @functools.partial(
        pl.kernel,
        out_type=jax.ShapeDtypeStruct((NC * N_PAD, D), jnp.float32),
        mesh=mesh,
        scratch_types=[
            pltpu.VMEM((epw,), jnp.uint32),
            pltpu.VMEM((2, CH), jnp.int32),
            pltpu.VMEM((2, CH), jnp.int32),
            pltpu.VMEM((CH, D), jnp.float32),
            pltpu.VMEM((CH, D), jnp.float32),
            pltpu.VMEM((2, CH * DH), jnp.uint32),
            pltpu.VMEM_SHARED((N_PAD, D), jnp.float32),
            pltpu.SemaphoreType.DMA,
            pltpu.SemaphoreType.DMA,
        ],
    )
    def edge_kernel(y_hbm, idx_hbm, fp_hbm, zeros_hbm, out_hbm,
                    pidx_v, idxj_r, idxi_r, rows0_v, rows1_v, fp_r,
                    acc_sh, sem0, sem1):
        c = lax.axis_index("c")
        s = lax.axis_index("s")
        wid = s * NC + c
        row0 = s * RPT
        base0 = wid * epw

        pltpu.sync_copy(zeros_hbm, rows0_v)
        for off, n in _RPT_CHUNKS:
            pltpu.sync_copy(rows0_v.at[pl.ds(0, n)],
                            acc_sh.at[pl.ds(row0 + off, n)])

        pltpu.sync_copy(idx_hbm.at[pl.ds(base0, epw)], pidx_v)
        plsc.subcore_barrier()

        bufs = ((rows0_v, sem0), (rows1_v, sem1))

        def unpack_idx(kb, b):
            off = jnp.minimum(kb, nchunk - 1) * CH
            for g in range(CH // 16):
                u = pidx_v[pl.ds(off + g * 16, 16)]
                sl = pl.ds(g * 16, 16)
                idxj_r[b, sl] = (u >> 16).astype(jnp.int32)
                idxi_r[b, sl] = (u & 0xFFFF).astype(jnp.int32)

        def issue(kb, b):
            kc = jnp.minimum(kb, nchunk - 1)
            rows, sem = bufs[b]
            pltpu.async_copy(y_hbm.at[idxj_r.at[b]], rows, sem)
            pltpu.async_copy(
                fp_hbm.at[pl.ds((base0 + kc * CH) * DH, CH * DH)],
                fp_r.at[b], sem)

        def drain(kb, b):
            kc = jnp.minimum(kb, nchunk - 1)
            rows, sem = bufs[b]
            pltpu.make_async_copy(y_hbm.at[idxj_r.at[b]], rows, sem).wait()
            pltpu.make_async_copy(
                fp_hbm.at[pl.ds((base0 + kc * CH) * DH, CH * DH)],
                fp_r.at[b], sem).wait()

        unpack_idx(0, 0)
        issue(0, 0)
        unpack_idx(1, 1)
        issue(1, 1)

        himask = jnp.uint32(0xFFFF0000)

        def pair_body(i, carry):
            k = i * 2
            for b in range(2):
                kb = k + b
                rows, sem = bufs[b]
                drain(kb, b)

                def mul_body(q, c2):
                    for g in range(D // 32):
                        uf = fp_r[b, pl.ds(q * DH + g * 16, 16)]
                        flo = lax.bitcast_convert_type(uf << 16, jnp.float32)
                        fhi = lax.bitcast_convert_type(uf & himask, jnp.float32)
                        lo_sl = pl.ds(g * 32, 16)
                        hi_sl = pl.ds(g * 32 + 16, 16)
                        rows[q, lo_sl] = rows[q, lo_sl] * flo
                        rows[q, hi_sl] = rows[q, hi_sl] * fhi
                    return c2

                lax.fori_loop(0, CH, mul_body, 0, unroll=4)
                pltpu.sync_copy(rows, acc_sh.at[idxi_r.at[b]], add=True)
                unpack_idx(kb + 2, b)
                issue(kb + 2, b)
            return carry

        lax.fori_loop(0, nchunk // 2, pair_body, 0)
        drain(nchunk, 0)
        drain(nchunk + 1, 1)
        plsc.subcore_barrier()

        out0 = c * N_PAD + row0
        for off, n in _RPT_CHUNKS:
            pltpu.sync_copy(acc_sh.at[pl.ds(row0 + off, n)],
                            rows0_v.at[pl.ds(0, n)])
            pltpu.sync_copy(rows0_v.at[pl.ds(0, n)],
                            out_hbm.at[pl.ds(out0 + off, n)])

    return edge_kernel


def _out_body(xi_ref, part_ref, emb_ref, w1T_ref, b1_ref, w2T_ref, b2_ref,
              wvT_ref, bv_ref, gate_ref, out_ref):
    m = xi_ref[...] + part_ref[0] + part_ref[1]
    for r in range(3):
        h = _softplus(
            jnp.dot(m, w1T_ref[r], preferred_element_type=jnp.float32)
            + b1_ref[r:r + 1, :])
        h = jnp.dot(h, w2T_ref[r], preferred_element_type=jnp.float32) \
            + b2_ref[r:r + 1, :]
        m = m + h
    v = jnp.dot(_softplus(m), wvT_ref[...],
                preferred_element_type=jnp.float32) + bv_ref[...]
    out_ref[...] = jax.nn.sigmoid(gate_ref[...]) * emb_ref[...] + v


def _out_stage(xi, parts3, emb, w1T, b1, w2T, b2, wvT, bv2, gate2):
    nb = 1000
    return pl.pallas_call(
        _out_body,
        grid=(N_NODES // nb,),
        in_specs=[
            pl.BlockSpec((nb, D), lambda i: (i, 0)),
            pl.BlockSpec((NC, nb, D), lambda i: (0, i, 0)),
            pl.BlockSpec((nb, D), lambda i: (i, 0)),
            pl.BlockSpec((3, D, D), lambda i: (0, 0, 0)),
            pl.BlockSpec((3, D), lambda i: (0, 0)),
            pl.BlockSpec((3, D, D), lambda i: (0, 0, 0)),
            pl.BlockSpec((3, D), lambda i: (0, 0)),
            pl.BlockSpec((D, D), lambda i: (0, 0)),
            pl.BlockSpec((1, D), lambda i: (0, 0)),
            pl.BlockSpec((1, D), lambda i: (0, 0)),
        ],
        out_specs=pl.BlockSpec((nb, D), lambda i: (i, 0)),
        out_shape=jax.ShapeDtypeStruct((N_NODES, D), jnp.float32),
    )(xi, parts3, emb, w1T, b1, w2T, b2, wvT, bv2, gate2)


def kernel(atomic_embedding, pair_indices, f_ij, d_ij, G, Wi, bi, Wj, bj,
           Wv, bv, res_W1, res_b1, res_W2, res_b2, gate):
    e = pair_indices.shape[1]
    npc = -(-e // (NW * CH))
    npc += npc % 2
    e_pad = NW * CH * npc
    pad = e_pad - e
    packed_idx = jnp.pad(
        (pair_indices[1].astype(jnp.uint32) << 16)
        | pair_indices[0].astype(jnp.uint32), (0, pad))
    f_pad = jnp.pad(f_ij, ((0, pad), (0, 0)))

    gT = G.T
    xi, y = _node_stage(atomic_embedding, Wi.T, bi.reshape(1, D),
                        Wj.T, bj.reshape(1, D))
    fpu = _fp_stage(f_pad, gT[:, _COLS_A], gT[:, _COLS_B], e_pad)

    zeros = jnp.zeros((CH, D), jnp.float32)
    parts = _make_edge_stage(e_pad)(y, packed_idx, fpu.reshape(e_pad * DH),
                                    zeros)
    parts3 = parts.reshape(NC, N_PAD, D)

    return _out_stage(xi, parts3, atomic_embedding,
                      res_W1.transpose(0, 2, 1), res_b1,
                      res_W2.transpose(0, 2, 1), res_b2,
                      Wv.T, bv.reshape(1, D), gate.reshape(1, D))


# speedup vs baseline: 1.1802x; 1.1802x over previous
import functools

import numpy as np
import jax
import jax.numpy as jnp
from jax import lax
from jax.experimental import pallas as pl
from jax.experimental.pallas import tpu as pltpu
from jax.experimental.pallas import tpu_sc as plsc

N_NODES = 10000
D = 128
DH = D // 2

NC = 2
NS = 16
NW = NC * NS
CH = 64
N_PAD = 10240
RPT = N_PAD // NS

_COLS_A = (np.arange(DH) // 16) * 32 + np.arange(DH) % 16
_COLS_B = _COLS_A + 16

_RPT_CHUNKS = [(t * CH, CH) for t in range(RPT // CH)]
if RPT % CH:
    _RPT_CHUNKS.append(((RPT // CH) * CH, RPT % CH))

_LOG2 = 0.6931471805599453


def _softplus(x):
    return jnp.logaddexp(x, 0.0)


def _pack_bf16_pairs(a, b):
    au = lax.bitcast_convert_type(a.astype(jnp.bfloat16), jnp.uint16)
    bu = lax.bitcast_convert_type(b.astype(jnp.bfloat16), jnp.uint16)
    return au.astype(jnp.uint32) | (bu.astype(jnp.uint32) << 16)


def _node_body(emb_ref, wiT_ref, bi_ref, wjT_ref, bj_ref, xi_ref, y_ref):
    x = _softplus(emb_ref[...]) - _LOG2
    xi_ref[...] = _softplus(
        jnp.dot(x, wiT_ref[...], preferred_element_type=jnp.float32) + bi_ref[...])
    y_ref[...] = _softplus(
        jnp.dot(x, wjT_ref[...], preferred_element_type=jnp.float32) + bj_ref[...])


def _node_stage(emb, wiT, bi2, wjT, bj2):
    nb = 1000
    return pl.pallas_call(
        _node_body,
        grid=(N_NODES // nb,),
        in_specs=[
            pl.BlockSpec((nb, D), lambda i: (i, 0)),
            pl.BlockSpec((D, D), lambda i: (0, 0)),
            pl.BlockSpec((1, D), lambda i: (0, 0)),
            pl.BlockSpec((D, D), lambda i: (0, 0)),
            pl.BlockSpec((1, D), lambda i: (0, 0)),
        ],
        out_specs=[pl.BlockSpec((nb, D), lambda i: (i, 0)),
                   pl.BlockSpec((nb, D), lambda i: (i, 0))],
        out_shape=[jax.ShapeDtypeStruct((N_NODES, D), jnp.float32),
                   jax.ShapeDtypeStruct((N_NODES, D), jnp.float32)],
    )(emb, wiT, bi2, wjT, bj2)


def _fp_body(f_ref, gTa_ref, gTb_ref, out_ref):
    fa = jnp.dot(f_ref[...], gTa_ref[...], preferred_element_type=jnp.float32)
    fb = jnp.dot(f_ref[...], gTb_ref[...], preferred_element_type=jnp.float32)
    out_ref[...] = _pack_bf16_pairs(fa, fb)


def _fp_stage(f_pad, gTa, gTb, e_pad):
    eb = 2048
    return pl.pallas_call(
        _fp_body,
        grid=(e_pad // eb,),
        in_specs=[pl.BlockSpec((eb, 16), lambda i: (i, 0)),
                  pl.BlockSpec((16, DH), lambda i: (0, 0)),
                  pl.BlockSpec((16, DH), lambda i: (0, 0))],
        out_specs=pl.BlockSpec((eb, DH), lambda i: (i, 0)),
        out_shape=jax.ShapeDtypeStruct((e_pad, DH), jnp.uint32),
    )(f_pad, gTa, gTb)


def _make_edge_stage(e_pad):
    epw = e_pad // NW
    nchunk = epw // CH

    mesh = plsc.VectorSubcoreMesh(core_axis_name="c", subcore_axis_name="s")

    @functools.partial(
        pl.kernel,
        out_type=jax.ShapeDtypeStruct((NC * N_PAD, D), jnp.float32),
        mesh=mesh,
        scratch_types=[
            pltpu.VMEM((epw,), jnp.uint32),
            pltpu.VMEM((2, CH), jnp.int32),
            pltpu.VMEM((2, CH), jnp.int32),
            pltpu.VMEM((CH, D), jnp.float32),
            pltpu.VMEM((CH, D), jnp.float32),
            pltpu.VMEM((2, CH // 8, 8 * DH), jnp.uint32),
            pltpu.VMEM_SHARED((N_PAD, D), jnp.float32),
            pltpu.SemaphoreType.DMA,
            pltpu.SemaphoreType.DMA,
        ],
    )
    def edge_kernel(y_hbm, idx_hbm, fp_hbm, zeros_hbm, out_hbm,
                    pidx_v, idxj_r, idxi_r, rows0_v, rows1_v, fp_r,
                    acc_sh, sem0, sem1):
        c = lax.axis_index("c")
        s = lax.axis_index("s")
        wid = s * NC + c
        row0 = s * RPT
        base0 = wid * epw

        pltpu.sync_copy(zeros_hbm, rows0_v)
        for off, n in _RPT_CHUNKS:
            pltpu.sync_copy(rows0_v.at[pl.ds(0, n)],
                            acc_sh.at[pl.ds(row0 + off, n)])

        pltpu.sync_copy(idx_hbm.at[pl.ds(base0, epw)], pidx_v)
        plsc.subcore_barrier()

        bufs = ((rows0_v, sem0), (rows1_v, sem1))

        def unpack_idx(kb, b):
            off = jnp.minimum(kb, nchunk - 1) * CH
            for g in range(CH // 16):
                u = pidx_v[pl.ds(off + g * 16, 16)]
                sl = pl.ds(g * 16, 16)
                idxj_r[b, sl] = (u >> 16).astype(jnp.int32)
                idxi_r[b, sl] = (u & 0xFFFF).astype(jnp.int32)

        def issue(kb, b):
            kc = jnp.minimum(kb, nchunk - 1)
            rows, sem = bufs[b]
            pltpu.async_copy(y_hbm.at[idxj_r.at[b]], rows, sem)
            pltpu.async_copy(
                fp_hbm.at[pl.ds(pl.multiple_of((base0 + kc * CH) // 8, 8),
                                CH // 8)],
                fp_r.at[b], sem)

        def drain(kb, b):
            kc = jnp.minimum(kb, nchunk - 1)
            rows, sem = bufs[b]
            pltpu.make_async_copy(y_hbm.at[idxj_r.at[b]], rows, sem).wait()
            pltpu.make_async_copy(
                fp_hbm.at[pl.ds(pl.multiple_of((base0 + kc * CH) // 8, 8),
                                CH // 8)],
                fp_r.at[b], sem).wait()

        unpack_idx(0, 0)
        issue(0, 0)
        unpack_idx(1, 1)
        issue(1, 1)

        himask = jnp.uint32(0xFFFF0000)

        def pair_body(i, carry):
            k = i * 2
            for b in range(2):
                kb = k + b
                rows, sem = bufs[b]
                drain(kb, b)

                def mul_body(q, c2):
                    fr = q >> 3
                    fc = (q & 7) * DH
                    ng = D // 32
                    ufs = [fp_r[b, fr, pl.ds(fc + g * 16, 16)]
                           for g in range(ng)]
                    ylos = [rows[q, pl.ds(g * 32, 16)] for g in range(ng)]
                    yhis = [rows[q, pl.ds(g * 32 + 16, 16)] for g in range(ng)]
                    plos = [y * lax.bitcast_convert_type(u << 16, jnp.float32)
                            for y, u in zip(ylos, ufs)]
                    phis = [y * lax.bitcast_convert_type(u & himask, jnp.float32)
                            for y, u in zip(yhis, ufs)]
                    for g in range(ng):
                        rows[q, pl.ds(g * 32, 16)] = plos[g]
                        rows[q, pl.ds(g * 32 + 16, 16)] = phis[g]
                    return c2

                lax.fori_loop(0, CH, mul_body, 0, unroll=4)
                pltpu.sync_copy(rows, acc_sh.at[idxi_r.at[b]], add=True)
                unpack_idx(kb + 2, b)
                issue(kb + 2, b)
            return carry

        lax.fori_loop(0, nchunk // 2, pair_body, 0)
        drain(nchunk, 0)
        drain(nchunk + 1, 1)
        plsc.subcore_barrier()

        out0 = c * N_PAD + row0
        for off, n in _RPT_CHUNKS:
            pltpu.sync_copy(acc_sh.at[pl.ds(row0 + off, n)],
                            rows0_v.at[pl.ds(0, n)])
            pltpu.sync_copy(rows0_v.at[pl.ds(0, n)],
                            out_hbm.at[pl.ds(out0 + off, n)])

    return edge_kernel


def _out_body(xi_ref, part_ref, emb_ref, w1T_ref, b1_ref, w2T_ref, b2_ref,
              wvT_ref, bv_ref, gate_ref, out_ref):
    m = xi_ref[...] + part_ref[0] + part_ref[1]
    for r in range(3):
        h = _softplus(
            jnp.dot(m, w1T_ref[r], preferred_element_type=jnp.float32)
            + b1_ref[r:r + 1, :])
        h = jnp.dot(h, w2T_ref[r], preferred_element_type=jnp.float32) \
            + b2_ref[r:r + 1, :]
        m = m + h
    v = jnp.dot(_softplus(m), wvT_ref[...],
                preferred_element_type=jnp.float32) + bv_ref[...]
    out_ref[...] = jax.nn.sigmoid(gate_ref[...]) * emb_ref[...] + v


def _out_stage(xi, parts3, emb, w1T, b1, w2T, b2, wvT, bv2, gate2):
    nb = 1000
    return pl.pallas_call(
        _out_body,
        grid=(N_NODES // nb,),
        in_specs=[
            pl.BlockSpec((nb, D), lambda i: (i, 0)),
            pl.BlockSpec((NC, nb, D), lambda i: (0, i, 0)),
            pl.BlockSpec((nb, D), lambda i: (i, 0)),
            pl.BlockSpec((3, D, D), lambda i: (0, 0, 0)),
            pl.BlockSpec((3, D), lambda i: (0, 0)),
            pl.BlockSpec((3, D, D), lambda i: (0, 0, 0)),
            pl.BlockSpec((3, D), lambda i: (0, 0)),
            pl.BlockSpec((D, D), lambda i: (0, 0)),
            pl.BlockSpec((1, D), lambda i: (0, 0)),
            pl.BlockSpec((1, D), lambda i: (0, 0)),
        ],
        out_specs=pl.BlockSpec((nb, D), lambda i: (i, 0)),
        out_shape=jax.ShapeDtypeStruct((N_NODES, D), jnp.float32),
    )(xi, parts3, emb, w1T, b1, w2T, b2, wvT, bv2, gate2)


def kernel(atomic_embedding, pair_indices, f_ij, d_ij, G, Wi, bi, Wj, bj,
           Wv, bv, res_W1, res_b1, res_W2, res_b2, gate):
    e = pair_indices.shape[1]
    npc = -(-e // (NW * CH))
    npc += npc % 2
    e_pad = NW * CH * npc
    pad = e_pad - e
    packed_idx = jnp.pad(
        (pair_indices[1].astype(jnp.uint32) << 16)
        | pair_indices[0].astype(jnp.uint32), (0, pad))
    f_pad = jnp.pad(f_ij, ((0, pad), (0, 0)))

    gT = G.T
    xi, y = _node_stage(atomic_embedding, Wi.T, bi.reshape(1, D),
                        Wj.T, bj.reshape(1, D))
    fpu = _fp_stage(f_pad, gT[:, _COLS_A], gT[:, _COLS_B], e_pad)

    zeros = jnp.zeros((CH, D), jnp.float32)
    parts = _make_edge_stage(e_pad)(y, packed_idx,
                                    fpu.reshape(e_pad // 8, 8 * DH), zeros)
    parts3 = parts.reshape(NC, N_PAD, D)

    return _out_stage(xi, parts3, atomic_embedding,
                      res_W1.transpose(0, 2, 1), res_b1,
                      res_W2.transpose(0, 2, 1), res_b2,
                      Wv.T, bv.reshape(1, D), gate.reshape(1, D))


# speedup vs baseline: 1.5600x; 1.3218x over previous
import functools

import numpy as np
import jax
import jax.numpy as jnp
from jax import lax
from jax.experimental import pallas as pl
from jax.experimental.pallas import tpu as pltpu
from jax.experimental.pallas import tpu_sc as plsc

N_NODES = 10000
D = 128
DH = D // 2

NC = 2
NS = 16
NW = NC * NS
CH = 64
N_PAD = 10240
RPT = N_PAD // NS

_COLS_A = (np.arange(DH) // 16) * 32 + np.arange(DH) % 16
_COLS_B = _COLS_A + 16

_RPT_CHUNKS = [(t * CH, CH) for t in range(RPT // CH)]
if RPT % CH:
    _RPT_CHUNKS.append(((RPT // CH) * CH, RPT % CH))

_LOG2 = 0.6931471805599453


def _softplus(x):
    return jnp.logaddexp(x, 0.0)


def _pack_bf16_pairs(a, b):
    au = lax.bitcast_convert_type(a.astype(jnp.bfloat16), jnp.uint16)
    bu = lax.bitcast_convert_type(b.astype(jnp.bfloat16), jnp.uint16)
    return au.astype(jnp.uint32) | (bu.astype(jnp.uint32) << 16)


def _node_body(emb_ref, wiT_ref, bi_ref, wjT_ref, bj_ref, xi_ref, y_ref):
    x = _softplus(emb_ref[...]) - _LOG2
    xi_ref[...] = _softplus(
        jnp.dot(x, wiT_ref[...], preferred_element_type=jnp.float32) + bi_ref[...])
    y_ref[...] = _softplus(
        jnp.dot(x, wjT_ref[...], preferred_element_type=jnp.float32) + bj_ref[...])


def _node_stage(emb, wiT, bi2, wjT, bj2):
    nb = 1000
    return pl.pallas_call(
        _node_body,
        grid=(N_NODES // nb,),
        in_specs=[
            pl.BlockSpec((nb, D), lambda i: (i, 0)),
            pl.BlockSpec((D, D), lambda i: (0, 0)),
            pl.BlockSpec((1, D), lambda i: (0, 0)),
            pl.BlockSpec((D, D), lambda i: (0, 0)),
            pl.BlockSpec((1, D), lambda i: (0, 0)),
        ],
        out_specs=[pl.BlockSpec((nb, D), lambda i: (i, 0)),
                   pl.BlockSpec((nb, D), lambda i: (i, 0))],
        out_shape=[jax.ShapeDtypeStruct((N_NODES, D), jnp.float32),
                   jax.ShapeDtypeStruct((N_NODES, D), jnp.float32)],
    )(emb, wiT, bi2, wjT, bj2)


_FPB = 256


def _fp_stage(f_wide, wa, wb, e_pad):
    nrows = f_wide.shape[0]
    last_blk = (nrows - 1) // _FPB

    def body(f_ref, wa_ref, wb_ref, out_ref):
        i = pl.program_id(0)
        fb = f_ref[...].astype(jnp.bfloat16)
        ma = jnp.dot(fb, wa_ref[...], preferred_element_type=jnp.float32)
        mb = jnp.dot(fb, wb_ref[...], preferred_element_type=jnp.float32)
        packed = _pack_bf16_pairs(ma, mb)
        row = lax.broadcasted_iota(jnp.int32, (_FPB, 8 * DH), 0) + i * _FPB
        out_ref[...] = jnp.where(row < nrows, packed, jnp.uint32(0))

    return pl.pallas_call(
        body,
        grid=(e_pad // 8 // _FPB,),
        in_specs=[
            pl.BlockSpec((_FPB, D), lambda i: (jnp.minimum(i, last_blk), 0)),
            pl.BlockSpec((D, 8 * DH), lambda i: (0, 0)),
            pl.BlockSpec((D, 8 * DH), lambda i: (0, 0)),
        ],
        out_specs=pl.BlockSpec((_FPB, 8 * DH), lambda i: (i, 0)),
        out_shape=jax.ShapeDtypeStruct((e_pad // 8, 8 * DH), jnp.uint32),
    )(f_wide, wa, wb)


def _make_edge_stage(e_pad):
    epw = e_pad // NW
    nchunk = epw // CH

    mesh = plsc.VectorSubcoreMesh(core_axis_name="c", subcore_axis_name="s")

    @functools.partial(
        pl.kernel,
        out_type=jax.ShapeDtypeStruct((NC * N_PAD, D), jnp.float32),
        mesh=mesh,
        scratch_types=[
            pltpu.VMEM((epw,), jnp.uint32),
            pltpu.VMEM((2, CH), jnp.int32),
            pltpu.VMEM((2, CH), jnp.int32),
            pltpu.VMEM((CH, D), jnp.float32),
            pltpu.VMEM((CH, D), jnp.float32),
            pltpu.VMEM((2, CH // 8, 8 * DH), jnp.uint32),
            pltpu.VMEM_SHARED((N_PAD, D), jnp.float32),
            pltpu.SemaphoreType.DMA,
            pltpu.SemaphoreType.DMA,
        ],
    )
    def edge_kernel(y_hbm, idx_hbm, fp_hbm, zeros_hbm, out_hbm,
                    pidx_v, idxj_r, idxi_r, rows0_v, rows1_v, fp_r,
                    acc_sh, sem0, sem1):
        c = lax.axis_index("c")
        s = lax.axis_index("s")
        wid = s * NC + c
        row0 = s * RPT
        base0 = wid * epw

        pltpu.sync_copy(zeros_hbm, rows0_v)
        for off, n in _RPT_CHUNKS:
            pltpu.sync_copy(rows0_v.at[pl.ds(0, n)],
                            acc_sh.at[pl.ds(row0 + off, n)])

        pltpu.sync_copy(idx_hbm.at[pl.ds(base0, epw)], pidx_v)
        plsc.subcore_barrier()

        bufs = ((rows0_v, sem0), (rows1_v, sem1))

        def unpack_idx(kb, b):
            off = jnp.minimum(kb, nchunk - 1) * CH
            for g in range(CH // 16):
                u = pidx_v[pl.ds(off + g * 16, 16)]
                sl = pl.ds(g * 16, 16)
                idxj_r[b, sl] = (u >> 16).astype(jnp.int32)
                idxi_r[b, sl] = (u & 0xFFFF).astype(jnp.int32)

        def issue(kb, b):
            kc = jnp.minimum(kb, nchunk - 1)
            rows, sem = bufs[b]
            pltpu.async_copy(y_hbm.at[idxj_r.at[b]], rows, sem)
            pltpu.async_copy(
                fp_hbm.at[pl.ds(pl.multiple_of((base0 + kc * CH) // 8, 8),
                                CH // 8)],
                fp_r.at[b], sem)

        def drain(kb, b):
            kc = jnp.minimum(kb, nchunk - 1)
            rows, sem = bufs[b]
            pltpu.make_async_copy(y_hbm.at[idxj_r.at[b]], rows, sem).wait()
            pltpu.make_async_copy(
                fp_hbm.at[pl.ds(pl.multiple_of((base0 + kc * CH) // 8, 8),
                                CH // 8)],
                fp_r.at[b], sem).wait()

        unpack_idx(0, 0)
        issue(0, 0)
        unpack_idx(1, 1)
        issue(1, 1)

        himask = jnp.uint32(0xFFFF0000)

        def pair_body(i, carry):
            k = i * 2
            for b in range(2):
                kb = k + b
                rows, sem = bufs[b]
                drain(kb, b)

                def mul_body(q, c2):
                    fr = q >> 3
                    fc = (q & 7) * DH
                    ng = D // 32
                    ufs = [fp_r[b, fr, pl.ds(fc + g * 16, 16)]
                           for g in range(ng)]
                    ylos = [rows[q, pl.ds(g * 32, 16)] for g in range(ng)]
                    yhis = [rows[q, pl.ds(g * 32 + 16, 16)] for g in range(ng)]
                    plos = [y * lax.bitcast_convert_type(u << 16, jnp.float32)
                            for y, u in zip(ylos, ufs)]
                    phis = [y * lax.bitcast_convert_type(u & himask, jnp.float32)
                            for y, u in zip(yhis, ufs)]
                    for g in range(ng):
                        rows[q, pl.ds(g * 32, 16)] = plos[g]
                        rows[q, pl.ds(g * 32 + 16, 16)] = phis[g]
                    return c2

                lax.fori_loop(0, CH, mul_body, 0, unroll=4)
                pltpu.sync_copy(rows, acc_sh.at[idxi_r.at[b]], add=True)
                unpack_idx(kb + 2, b)
                issue(kb + 2, b)
            return carry

        lax.fori_loop(0, nchunk // 2, pair_body, 0)
        drain(nchunk, 0)
        drain(nchunk + 1, 1)
        plsc.subcore_barrier()

        out0 = c * N_PAD + row0
        for off, n in _RPT_CHUNKS:
            pltpu.sync_copy(acc_sh.at[pl.ds(row0 + off, n)],
                            rows0_v.at[pl.ds(0, n)])
            pltpu.sync_copy(rows0_v.at[pl.ds(0, n)],
                            out_hbm.at[pl.ds(out0 + off, n)])

    return edge_kernel


def _out_body(xi_ref, part_ref, emb_ref, w1T_ref, b1_ref, w2T_ref, b2_ref,
              wvT_ref, bv_ref, gate_ref, out_ref):
    m = xi_ref[...] + part_ref[0] + part_ref[1]
    for r in range(3):
        h = _softplus(
            jnp.dot(m, w1T_ref[r], preferred_element_type=jnp.float32)
            + b1_ref[r:r + 1, :])
        h = jnp.dot(h, w2T_ref[r], preferred_element_type=jnp.float32) \
            + b2_ref[r:r + 1, :]
        m = m + h
    v = jnp.dot(_softplus(m), wvT_ref[...],
                preferred_element_type=jnp.float32) + bv_ref[...]
    out_ref[...] = jax.nn.sigmoid(gate_ref[...]) * emb_ref[...] + v


def _out_stage(xi, parts3, emb, w1T, b1, w2T, b2, wvT, bv2, gate2):
    nb = 1000
    return pl.pallas_call(
        _out_body,
        grid=(N_NODES // nb,),
        in_specs=[
            pl.BlockSpec((nb, D), lambda i: (i, 0)),
            pl.BlockSpec((NC, nb, D), lambda i: (0, i, 0)),
            pl.BlockSpec((nb, D), lambda i: (i, 0)),
            pl.BlockSpec((3, D, D), lambda i: (0, 0, 0)),
            pl.BlockSpec((3, D), lambda i: (0, 0)),
            pl.BlockSpec((3, D, D), lambda i: (0, 0, 0)),
            pl.BlockSpec((3, D), lambda i: (0, 0)),
            pl.BlockSpec((D, D), lambda i: (0, 0)),
            pl.BlockSpec((1, D), lambda i: (0, 0)),
            pl.BlockSpec((1, D), lambda i: (0, 0)),
        ],
        out_specs=pl.BlockSpec((nb, D), lambda i: (i, 0)),
        out_shape=jax.ShapeDtypeStruct((N_NODES, D), jnp.float32),
    )(xi, parts3, emb, w1T, b1, w2T, b2, wvT, bv2, gate2)


def kernel(atomic_embedding, pair_indices, f_ij, d_ij, G, Wi, bi, Wj, bj,
           Wv, bv, res_W1, res_b1, res_W2, res_b2, gate):
    e = pair_indices.shape[1]
    npc = -(-e // (NW * CH))
    npc += npc % 2
    e_pad = NW * CH * npc
    pad = e_pad - e
    packed_idx = jnp.pad(
        (pair_indices[1].astype(jnp.uint32) << 16)
        | pair_indices[0].astype(jnp.uint32), (0, pad))
    f_wide = f_ij.reshape(e // 8, 8 * 16)

    gT = G.T
    eye8 = jnp.eye(8, dtype=jnp.float32)
    wa = jnp.kron(eye8, gT[:, _COLS_A]).astype(jnp.bfloat16)
    wb = jnp.kron(eye8, gT[:, _COLS_B]).astype(jnp.bfloat16)
    xi, y = _node_stage(atomic_embedding, Wi.T, bi.reshape(1, D),
                        Wj.T, bj.reshape(1, D))
    fpu = _fp_stage(f_wide, wa, wb, e_pad)

    zeros = jnp.zeros((CH, D), jnp.float32)
    parts = _make_edge_stage(e_pad)(y, packed_idx, fpu, zeros)
    parts3 = parts.reshape(NC, N_PAD, D)

    return _out_stage(xi, parts3, atomic_embedding,
                      res_W1.transpose(0, 2, 1), res_b1,
                      res_W2.transpose(0, 2, 1), res_b2,
                      Wv.T, bv.reshape(1, D), gate.reshape(1, D))


# speedup vs baseline: 1.6592x; 1.0636x over previous
import functools

import numpy as np
import jax
import jax.numpy as jnp
from jax import lax
from jax.experimental import pallas as pl
from jax.experimental.pallas import tpu as pltpu
from jax.experimental.pallas import tpu_sc as plsc

N_NODES = 10000
D = 128
DH = D // 2

NC = 2
NS = 16
NW = NC * NS
CH = 64
N_PAD = 10240
RPT = N_PAD // NS

_COLS_A = (np.arange(DH) // 16) * 32 + np.arange(DH) % 16
_COLS_B = _COLS_A + 16

_RPT_CHUNKS = [(t * CH, CH) for t in range(RPT // CH)]
if RPT % CH:
    _RPT_CHUNKS.append(((RPT // CH) * CH, RPT % CH))

_LOG2 = 0.6931471805599453


def _softplus(x):
    return jnp.logaddexp(x, 0.0)


def _pack_bf16_pairs(a, b):
    au = lax.bitcast_convert_type(a.astype(jnp.bfloat16), jnp.uint16)
    bu = lax.bitcast_convert_type(b.astype(jnp.bfloat16), jnp.uint16)
    return au.astype(jnp.uint32) | (bu.astype(jnp.uint32) << 16)


def _node_body(emb_ref, wiT_ref, bi_ref, wjT_ref, bj_ref, xi_ref, y_ref):
    x = _softplus(emb_ref[...]) - _LOG2
    xi_ref[...] = _softplus(
        jnp.dot(x, wiT_ref[...], preferred_element_type=jnp.float32) + bi_ref[...])
    y_ref[...] = _softplus(
        jnp.dot(x, wjT_ref[...], preferred_element_type=jnp.float32) + bj_ref[...])


def _node_stage(emb, wiT, bi2, wjT, bj2):
    nb = 1000
    return pl.pallas_call(
        _node_body,
        grid=(N_NODES // nb,),
        in_specs=[
            pl.BlockSpec((nb, D), lambda i: (i, 0)),
            pl.BlockSpec((D, D), lambda i: (0, 0)),
            pl.BlockSpec((1, D), lambda i: (0, 0)),
            pl.BlockSpec((D, D), lambda i: (0, 0)),
            pl.BlockSpec((1, D), lambda i: (0, 0)),
        ],
        out_specs=[pl.BlockSpec((nb, D), lambda i: (i, 0)),
                   pl.BlockSpec((nb, D), lambda i: (i, 0))],
        out_shape=[jax.ShapeDtypeStruct((N_NODES, D), jnp.float32),
                   jax.ShapeDtypeStruct((N_NODES, D), jnp.float32)],
    )(emb, wiT, bi2, wjT, bj2)


_FPB = 256


def _fp_stage(f_wide, wa, wb, e_pad):
    nrows = f_wide.shape[0]
    last_blk = (nrows - 1) // _FPB

    def body(f_ref, wa_ref, wb_ref, out_ref):
        i = pl.program_id(0)
        fb = f_ref[...]
        ma = jnp.dot(fb, wa_ref[...], preferred_element_type=jnp.float32)
        mb = jnp.dot(fb, wb_ref[...], preferred_element_type=jnp.float32)
        packed = _pack_bf16_pairs(ma, mb)
        row = lax.broadcasted_iota(jnp.int32, (_FPB, 8 * DH), 0) + i * _FPB
        out_ref[...] = jnp.where(row < nrows, packed, jnp.uint32(0))

    return pl.pallas_call(
        body,
        grid=(e_pad // 8 // _FPB,),
        in_specs=[
            pl.BlockSpec((_FPB, D), lambda i: (jnp.minimum(i, last_blk), 0)),
            pl.BlockSpec((D, 8 * DH), lambda i: (0, 0)),
            pl.BlockSpec((D, 8 * DH), lambda i: (0, 0)),
        ],
        out_specs=pl.BlockSpec((_FPB, 8 * DH), lambda i: (i, 0)),
        out_shape=jax.ShapeDtypeStruct((e_pad // 8, 8 * DH), jnp.uint32),
    )(f_wide, wa, wb)


def _make_edge_stage(e_pad):
    epw = e_pad // NW
    nchunk = epw // CH
    cnt0 = max(2, int(2 * nchunk * 0.61) // 2 * 2)
    cnt1 = 2 * nchunk - cnt0

    mesh = plsc.VectorSubcoreMesh(core_axis_name="c", subcore_axis_name="s")

    @functools.partial(
        pl.kernel,
        out_type=jax.ShapeDtypeStruct((NC * N_PAD, D), jnp.float32),
        mesh=mesh,
        scratch_types=[
            pltpu.VMEM((cnt0 * CH,), jnp.uint32),
            pltpu.VMEM((2, CH), jnp.int32),
            pltpu.VMEM((2, CH), jnp.int32),
            pltpu.VMEM((CH, D), jnp.float32),
            pltpu.VMEM((CH, D), jnp.float32),
            pltpu.VMEM((2, CH // 8, 8 * DH), jnp.uint32),
            pltpu.VMEM_SHARED((N_PAD, D), jnp.float32),
            pltpu.SemaphoreType.DMA,
            pltpu.SemaphoreType.DMA,
        ],
    )
    def edge_kernel(y_hbm, idx_hbm, fp_hbm, zeros_hbm, out_hbm,
                    pidx_v, idxj_r, idxi_r, rows0_v, rows1_v, fp_r,
                    acc_sh, sem0, sem1):
        c = lax.axis_index("c")
        s = lax.axis_index("s")
        row0 = s * RPT
        nch = jnp.where(c == 0, cnt0, cnt1)
        base0 = jnp.where(c == 0, s * cnt0, NS * cnt0 + s * cnt1) * CH

        pltpu.sync_copy(zeros_hbm, rows0_v)
        for off, n in _RPT_CHUNKS:
            pltpu.sync_copy(rows0_v.at[pl.ds(0, n)],
                            acc_sh.at[pl.ds(row0 + off, n)])

        pltpu.sync_copy(idx_hbm.at[pl.ds(base0, cnt0 * CH)], pidx_v)
        plsc.subcore_barrier()

        bufs = ((rows0_v, sem0), (rows1_v, sem1))

        def unpack_idx(kb, b):
            off = jnp.minimum(kb, nch - 1) * CH
            for g in range(CH // 16):
                u = pidx_v[pl.ds(off + g * 16, 16)]
                sl = pl.ds(g * 16, 16)
                idxj_r[b, sl] = (u >> 16).astype(jnp.int32)
                idxi_r[b, sl] = (u & 0xFFFF).astype(jnp.int32)

        def issue(kb, b):
            kc = jnp.minimum(kb, nch - 1)
            rows, sem = bufs[b]
            pltpu.async_copy(y_hbm.at[idxj_r.at[b]], rows, sem)
            pltpu.async_copy(
                fp_hbm.at[pl.ds(pl.multiple_of((base0 + kc * CH) // 8, 8),
                                CH // 8)],
                fp_r.at[b], sem)

        def drain(kb, b):
            kc = jnp.minimum(kb, nch - 1)
            rows, sem = bufs[b]
            pltpu.make_async_copy(y_hbm.at[idxj_r.at[b]], rows, sem).wait()
            pltpu.make_async_copy(
                fp_hbm.at[pl.ds(pl.multiple_of((base0 + kc * CH) // 8, 8),
                                CH // 8)],
                fp_r.at[b], sem).wait()

        unpack_idx(0, 0)
        issue(0, 0)
        unpack_idx(1, 1)
        issue(1, 1)

        himask = jnp.uint32(0xFFFF0000)

        def pair_body(i, carry):
            k = i * 2
            for b in range(2):
                kb = k + b
                rows, sem = bufs[b]
                drain(kb, b)

                def mul_body(q, c2):
                    fr = q >> 3
                    fc = (q & 7) * DH
                    ng = D // 32
                    ufs = [fp_r[b, fr, pl.ds(fc + g * 16, 16)]
                           for g in range(ng)]
                    ylos = [rows[q, pl.ds(g * 32, 16)] for g in range(ng)]
                    yhis = [rows[q, pl.ds(g * 32 + 16, 16)] for g in range(ng)]
                    plos = [y * lax.bitcast_convert_type(u << 16, jnp.float32)
                            for y, u in zip(ylos, ufs)]
                    phis = [y * lax.bitcast_convert_type(u & himask, jnp.float32)
                            for y, u in zip(yhis, ufs)]
                    for g in range(ng):
                        rows[q, pl.ds(g * 32, 16)] = plos[g]
                        rows[q, pl.ds(g * 32 + 16, 16)] = phis[g]
                    return c2

                lax.fori_loop(0, CH, mul_body, 0, unroll=4)
                pltpu.sync_copy(rows, acc_sh.at[idxi_r.at[b]], add=True)
                unpack_idx(kb + 2, b)
                issue(kb + 2, b)
            return carry

        lax.fori_loop(0, nch // 2, pair_body, 0)
        drain(nch, 0)
        drain(nch + 1, 1)
        plsc.subcore_barrier()

        out0 = c * N_PAD + row0
        for off, n in _RPT_CHUNKS:
            pltpu.sync_copy(acc_sh.at[pl.ds(row0 + off, n)],
                            rows0_v.at[pl.ds(0, n)])
            pltpu.sync_copy(rows0_v.at[pl.ds(0, n)],
                            out_hbm.at[pl.ds(out0 + off, n)])

    return edge_kernel


def _out_body(xi_ref, part_ref, emb_ref, w1T_ref, b1_ref, w2T_ref, b2_ref,
              wvT_ref, bv_ref, gate_ref, out_ref):
    m = xi_ref[...] + part_ref[0] + part_ref[1]
    for r in range(3):
        h = _softplus(
            jnp.dot(m, w1T_ref[r], preferred_element_type=jnp.float32)
            + b1_ref[r:r + 1, :])
        h = jnp.dot(h, w2T_ref[r], preferred_element_type=jnp.float32) \
            + b2_ref[r:r + 1, :]
        m = m + h
    v = jnp.dot(_softplus(m), wvT_ref[...],
                preferred_element_type=jnp.float32) + bv_ref[...]
    out_ref[...] = jax.nn.sigmoid(gate_ref[...]) * emb_ref[...] + v


def _out_stage(xi, parts3, emb, w1T, b1, w2T, b2, wvT, bv2, gate2):
    nb = 1000
    return pl.pallas_call(
        _out_body,
        grid=(N_NODES // nb,),
        in_specs=[
            pl.BlockSpec((nb, D), lambda i: (i, 0)),
            pl.BlockSpec((NC, nb, D), lambda i: (0, i, 0)),
            pl.BlockSpec((nb, D), lambda i: (i, 0)),
            pl.BlockSpec((3, D, D), lambda i: (0, 0, 0)),
            pl.BlockSpec((3, D), lambda i: (0, 0)),
            pl.BlockSpec((3, D, D), lambda i: (0, 0, 0)),
            pl.BlockSpec((3, D), lambda i: (0, 0)),
            pl.BlockSpec((D, D), lambda i: (0, 0)),
            pl.BlockSpec((1, D), lambda i: (0, 0)),
            pl.BlockSpec((1, D), lambda i: (0, 0)),
        ],
        out_specs=pl.BlockSpec((nb, D), lambda i: (i, 0)),
        out_shape=jax.ShapeDtypeStruct((N_NODES, D), jnp.float32),
    )(xi, parts3, emb, w1T, b1, w2T, b2, wvT, bv2, gate2)


def kernel(atomic_embedding, pair_indices, f_ij, d_ij, G, Wi, bi, Wj, bj,
           Wv, bv, res_W1, res_b1, res_W2, res_b2, gate):
    e = pair_indices.shape[1]
    npc = -(-e // (NW * CH))
    npc += npc % 2
    e_pad = NW * CH * npc
    pad = e_pad - e
    npc0 = max(2, int(2 * npc * 0.61) // 2 * 2)
    packed_idx = jnp.pad(
        (pair_indices[1].astype(jnp.uint32) << 16)
        | pair_indices[0].astype(jnp.uint32),
        (0, pad + (2 * npc0 - 2 * npc) * CH))
    f_wide = f_ij.reshape(e // 8, 8 * 16).astype(jnp.bfloat16)

    gT = G.T
    eye8 = jnp.eye(8, dtype=jnp.float32)
    wa = jnp.kron(eye8, gT[:, _COLS_A]).astype(jnp.bfloat16)
    wb = jnp.kron(eye8, gT[:, _COLS_B]).astype(jnp.bfloat16)
    xi, y = _node_stage(atomic_embedding, Wi.T, bi.reshape(1, D),
                        Wj.T, bj.reshape(1, D))
    fpu = _fp_stage(f_wide, wa, wb, e_pad)

    zeros = jnp.zeros((CH, D), jnp.float32)
    parts = _make_edge_stage(e_pad)(y, packed_idx, fpu, zeros)
    parts3 = parts.reshape(NC, N_PAD, D)

    return _out_stage(xi, parts3, atomic_embedding,
                      res_W1.transpose(0, 2, 1), res_b1,
                      res_W2.transpose(0, 2, 1), res_b2,
                      Wv.T, bv.reshape(1, D), gate.reshape(1, D))


# speedup vs baseline: 1.8091x; 1.0903x over previous
import functools

import numpy as np
import jax
import jax.numpy as jnp
from jax import lax
from jax.experimental import pallas as pl
from jax.experimental.pallas import tpu as pltpu
from jax.experimental.pallas import tpu_sc as plsc

N_NODES = 10000
D = 128
DH = D // 2

NC = 2
NS = 16
NW = NC * NS
CH = 64
N_PAD = 10240
RPT = N_PAD // NS

_COLS_A = (np.arange(DH) // 16) * 32 + np.arange(DH) % 16
_COLS_B = _COLS_A + 16

_RPT_CHUNKS = [(t * CH, CH) for t in range(RPT // CH)]
if RPT % CH:
    _RPT_CHUNKS.append(((RPT // CH) * CH, RPT % CH))

_LOG2 = 0.6931471805599453


def _softplus(x):
    return jnp.logaddexp(x, 0.0)


def _pack_bf16_pairs(a, b):
    au = lax.bitcast_convert_type(a.astype(jnp.bfloat16), jnp.uint16)
    bu = lax.bitcast_convert_type(b.astype(jnp.bfloat16), jnp.uint16)
    return au.astype(jnp.uint32) | (bu.astype(jnp.uint32) << 16)


def _node_body(emb_ref, wiT_ref, bi_ref, wjT_ref, bj_ref, xi_ref, y_ref):
    x = _softplus(emb_ref[...]) - _LOG2
    xi_ref[...] = _softplus(
        jnp.dot(x, wiT_ref[...], preferred_element_type=jnp.float32) + bi_ref[...])
    y_ref[...] = _softplus(
        jnp.dot(x, wjT_ref[...], preferred_element_type=jnp.float32) + bj_ref[...])


def _node_stage(emb, wiT, bi2, wjT, bj2):
    nb = 1000
    return pl.pallas_call(
        _node_body,
        grid=(N_NODES // nb,),
        in_specs=[
            pl.BlockSpec((nb, D), lambda i: (i, 0)),
            pl.BlockSpec((D, D), lambda i: (0, 0)),
            pl.BlockSpec((1, D), lambda i: (0, 0)),
            pl.BlockSpec((D, D), lambda i: (0, 0)),
            pl.BlockSpec((1, D), lambda i: (0, 0)),
        ],
        out_specs=[pl.BlockSpec((nb, D), lambda i: (i, 0)),
                   pl.BlockSpec((nb, D), lambda i: (i, 0))],
        out_shape=[jax.ShapeDtypeStruct((N_NODES, D), jnp.float32),
                   jax.ShapeDtypeStruct((N_NODES, D), jnp.float32)],
    )(emb, wiT, bi2, wjT, bj2)


_FPB = 512


def _fp_stage(f_wide, wa, wb, e_pad):
    nrows = f_wide.shape[0]
    last_blk = (nrows - 1) // _FPB

    def body(f_ref, wa_ref, wb_ref, out_ref):
        i = pl.program_id(0)
        fb = f_ref[...]
        ma = jnp.dot(fb, wa_ref[...], preferred_element_type=jnp.float32)
        mb = jnp.dot(fb, wb_ref[...], preferred_element_type=jnp.float32)
        au = lax.bitcast_convert_type(ma, jnp.uint32) + 0x8000
        bu = lax.bitcast_convert_type(mb, jnp.uint32) + 0x8000
        out_ref[...] = (bu & jnp.uint32(0xFFFF0000)) | (au >> 16)

        @pl.when(i >= nrows // _FPB)
        def _mask_tail():
            row = lax.broadcasted_iota(jnp.int32, (_FPB, 8 * DH), 0) + i * _FPB
            out_ref[...] = jnp.where(row < nrows, out_ref[...], jnp.uint32(0))

    return pl.pallas_call(
        body,
        grid=(e_pad // 8 // _FPB,),
        in_specs=[
            pl.BlockSpec((_FPB, D), lambda i: (jnp.minimum(i, last_blk), 0)),
            pl.BlockSpec((D, 8 * DH), lambda i: (0, 0)),
            pl.BlockSpec((D, 8 * DH), lambda i: (0, 0)),
        ],
        out_specs=pl.BlockSpec((_FPB, 8 * DH), lambda i: (i, 0)),
        out_shape=jax.ShapeDtypeStruct((e_pad // 8, 8 * DH), jnp.uint32),
    )(f_wide, wa, wb)


def _make_edge_stage(e_pad):
    epw = e_pad // NW
    nchunk = epw // CH
    cnt0 = max(2, int(2 * nchunk * 0.63) // 2 * 2)
    cnt1 = 2 * nchunk - cnt0

    mesh = plsc.VectorSubcoreMesh(core_axis_name="c", subcore_axis_name="s")

    @functools.partial(
        pl.kernel,
        out_type=jax.ShapeDtypeStruct((NC * N_PAD, D), jnp.float32),
        mesh=mesh,
        scratch_types=[
            pltpu.VMEM((cnt0 * CH,), jnp.uint32),
            pltpu.VMEM((2, CH), jnp.int32),
            pltpu.VMEM((2, CH), jnp.int32),
            pltpu.VMEM((CH, D), jnp.float32),
            pltpu.VMEM((CH, D), jnp.float32),
            pltpu.VMEM((2, CH // 8, 8 * DH), jnp.uint32),
            pltpu.VMEM_SHARED((N_PAD, D), jnp.float32),
            pltpu.SemaphoreType.DMA,
            pltpu.SemaphoreType.DMA,
        ],
    )
    def edge_kernel(y_hbm, idx_hbm, fp_hbm, zeros_hbm, out_hbm,
                    pidx_v, idxj_r, idxi_r, rows0_v, rows1_v, fp_r,
                    acc_sh, sem0, sem1):
        c = lax.axis_index("c")
        s = lax.axis_index("s")
        row0 = s * RPT
        nch = jnp.where(c == 0, cnt0, cnt1)
        base0 = jnp.where(c == 0, s * cnt0, NS * cnt0 + s * cnt1) * CH

        pltpu.sync_copy(zeros_hbm, rows0_v)
        for off, n in _RPT_CHUNKS:
            pltpu.sync_copy(rows0_v.at[pl.ds(0, n)],
                            acc_sh.at[pl.ds(row0 + off, n)])

        pltpu.sync_copy(idx_hbm.at[pl.ds(base0, cnt0 * CH)], pidx_v)
        plsc.subcore_barrier()

        bufs = ((rows0_v, sem0), (rows1_v, sem1))

        def unpack_idx(kb, b):
            off = jnp.minimum(kb, nch - 1) * CH
            for g in range(CH // 16):
                u = pidx_v[pl.ds(off + g * 16, 16)]
                sl = pl.ds(g * 16, 16)
                idxj_r[b, sl] = (u >> 16).astype(jnp.int32)
                idxi_r[b, sl] = (u & 0xFFFF).astype(jnp.int32)

        def issue(kb, b):
            kc = jnp.minimum(kb, nch - 1)
            rows, sem = bufs[b]
            pltpu.async_copy(y_hbm.at[idxj_r.at[b]], rows, sem)
            pltpu.async_copy(
                fp_hbm.at[pl.ds(pl.multiple_of((base0 + kc * CH) // 8, 8),
                                CH // 8)],
                fp_r.at[b], sem)

        def drain(kb, b):
            kc = jnp.minimum(kb, nch - 1)
            rows, sem = bufs[b]
            pltpu.make_async_copy(y_hbm.at[idxj_r.at[b]], rows, sem).wait()
            pltpu.make_async_copy(
                fp_hbm.at[pl.ds(pl.multiple_of((base0 + kc * CH) // 8, 8),
                                CH // 8)],
                fp_r.at[b], sem).wait()

        unpack_idx(0, 0)
        issue(0, 0)
        unpack_idx(1, 1)
        issue(1, 1)

        himask = jnp.uint32(0xFFFF0000)

        def pair_body(i, carry):
            k = i * 2
            for b in range(2):
                kb = k + b
                rows, sem = bufs[b]
                drain(kb, b)

                def mul_body(q, c2):
                    fr = q >> 3
                    fc = (q & 7) * DH
                    ng = D // 32
                    ufs = [fp_r[b, fr, pl.ds(fc + g * 16, 16)]
                           for g in range(ng)]
                    ylos = [rows[q, pl.ds(g * 32, 16)] for g in range(ng)]
                    yhis = [rows[q, pl.ds(g * 32 + 16, 16)] for g in range(ng)]
                    plos = [y * lax.bitcast_convert_type(u << 16, jnp.float32)
                            for y, u in zip(ylos, ufs)]
                    phis = [y * lax.bitcast_convert_type(u & himask, jnp.float32)
                            for y, u in zip(yhis, ufs)]
                    for g in range(ng):
                        rows[q, pl.ds(g * 32, 16)] = plos[g]
                        rows[q, pl.ds(g * 32 + 16, 16)] = phis[g]
                    return c2

                lax.fori_loop(0, CH, mul_body, 0, unroll=4)
                pltpu.sync_copy(rows, acc_sh.at[idxi_r.at[b]], add=True)
                unpack_idx(kb + 2, b)
                issue(kb + 2, b)
            return carry

        lax.fori_loop(0, nch // 2, pair_body, 0)
        drain(nch, 0)
        drain(nch + 1, 1)
        plsc.subcore_barrier()

        out0 = c * N_PAD + row0
        for off, n in _RPT_CHUNKS:
            pltpu.sync_copy(acc_sh.at[pl.ds(row0 + off, n)],
                            rows0_v.at[pl.ds(0, n)])
            pltpu.sync_copy(rows0_v.at[pl.ds(0, n)],
                            out_hbm.at[pl.ds(out0 + off, n)])

    return edge_kernel


def _out_body(xi_ref, part_ref, emb_ref, w1T_ref, b1_ref, w2T_ref, b2_ref,
              wvT_ref, bv_ref, gate_ref, out_ref):
    m = xi_ref[...] + part_ref[0] + part_ref[1]
    for r in range(3):
        h = _softplus(
            jnp.dot(m, w1T_ref[r], preferred_element_type=jnp.float32)
            + b1_ref[r:r + 1, :])
        h = jnp.dot(h, w2T_ref[r], preferred_element_type=jnp.float32) \
            + b2_ref[r:r + 1, :]
        m = m + h
    v = jnp.dot(_softplus(m), wvT_ref[...],
                preferred_element_type=jnp.float32) + bv_ref[...]
    out_ref[...] = jax.nn.sigmoid(gate_ref[...]) * emb_ref[...] + v


def _out_stage(xi, parts3, emb, w1T, b1, w2T, b2, wvT, bv2, gate2):
    nb = 1000
    return pl.pallas_call(
        _out_body,
        grid=(N_NODES // nb,),
        in_specs=[
            pl.BlockSpec((nb, D), lambda i: (i, 0)),
            pl.BlockSpec((NC, nb, D), lambda i: (0, i, 0)),
            pl.BlockSpec((nb, D), lambda i: (i, 0)),
            pl.BlockSpec((3, D, D), lambda i: (0, 0, 0)),
            pl.BlockSpec((3, D), lambda i: (0, 0)),
            pl.BlockSpec((3, D, D), lambda i: (0, 0, 0)),
            pl.BlockSpec((3, D), lambda i: (0, 0)),
            pl.BlockSpec((D, D), lambda i: (0, 0)),
            pl.BlockSpec((1, D), lambda i: (0, 0)),
            pl.BlockSpec((1, D), lambda i: (0, 0)),
        ],
        out_specs=pl.BlockSpec((nb, D), lambda i: (i, 0)),
        out_shape=jax.ShapeDtypeStruct((N_NODES, D), jnp.float32),
    )(xi, parts3, emb, w1T, b1, w2T, b2, wvT, bv2, gate2)


def kernel(atomic_embedding, pair_indices, f_ij, d_ij, G, Wi, bi, Wj, bj,
           Wv, bv, res_W1, res_b1, res_W2, res_b2, gate):
    e = pair_indices.shape[1]
    npc = -(-e // (NW * CH))
    npc += npc % 2
    e_pad = NW * CH * npc
    pad = e_pad - e
    npc0 = max(2, int(2 * npc * 0.63) // 2 * 2)
    packed_idx = jnp.pad(
        (pair_indices[1].astype(jnp.uint32) << 16)
        | pair_indices[0].astype(jnp.uint32),
        (0, pad + (2 * npc0 - 2 * npc) * CH))
    f_wide = f_ij.reshape(e // 8, 8 * 16).astype(jnp.bfloat16)

    gT = G.T
    eye8 = jnp.eye(8, dtype=jnp.float32)
    wa = jnp.kron(eye8, gT[:, _COLS_A]).astype(jnp.bfloat16)
    wb = jnp.kron(eye8, gT[:, _COLS_B]).astype(jnp.bfloat16)
    xi, y = _node_stage(atomic_embedding, Wi.T, bi.reshape(1, D),
                        Wj.T, bj.reshape(1, D))
    fpu = _fp_stage(f_wide, wa, wb, e_pad)

    zeros = jnp.zeros((CH, D), jnp.float32)
    parts = _make_edge_stage(e_pad)(y, packed_idx, fpu, zeros)
    parts3 = parts.reshape(NC, N_PAD, D)

    return _out_stage(xi, parts3, atomic_embedding,
                      res_W1.transpose(0, 2, 1), res_b1,
                      res_W2.transpose(0, 2, 1), res_b2,
                      Wv.T, bv.reshape(1, D), gate.reshape(1, D))


# speedup vs baseline: 1.8275x; 1.0102x over previous
import functools

import numpy as np
import jax
import jax.numpy as jnp
from jax import lax
from jax.experimental import pallas as pl
from jax.experimental.pallas import tpu as pltpu
from jax.experimental.pallas import tpu_sc as plsc

N_NODES = 10000
D = 128
DH = D // 2

NC = 2
NS = 16
NW = NC * NS
CH = 64
N_PAD = 10240
RPT = N_PAD // NS

_COLS_A = (np.arange(DH) // 16) * 32 + np.arange(DH) % 16
_COLS_B = _COLS_A + 16

_RPT_CHUNKS = [(t * CH, CH) for t in range(RPT // CH)]
if RPT % CH:
    _RPT_CHUNKS.append(((RPT // CH) * CH, RPT % CH))

_LOG2 = 0.6931471805599453


def _softplus(x):
    return jnp.logaddexp(x, 0.0)


def _pack_bf16_pairs(a, b):
    au = lax.bitcast_convert_type(a.astype(jnp.bfloat16), jnp.uint16)
    bu = lax.bitcast_convert_type(b.astype(jnp.bfloat16), jnp.uint16)
    return au.astype(jnp.uint32) | (bu.astype(jnp.uint32) << 16)


def _node_body(emb_ref, wiT_ref, bi_ref, wjT_ref, bj_ref, xi_ref, y_ref):
    x = _softplus(emb_ref[...]) - _LOG2
    xi_ref[...] = _softplus(
        jnp.dot(x, wiT_ref[...], preferred_element_type=jnp.float32) + bi_ref[...])
    y_ref[...] = _softplus(
        jnp.dot(x, wjT_ref[...], preferred_element_type=jnp.float32) + bj_ref[...])


def _node_stage(emb, wiT, bi2, wjT, bj2):
    nb = 1000
    return pl.pallas_call(
        _node_body,
        grid=(N_NODES // nb,),
        in_specs=[
            pl.BlockSpec((nb, D), lambda i: (i, 0)),
            pl.BlockSpec((D, D), lambda i: (0, 0)),
            pl.BlockSpec((1, D), lambda i: (0, 0)),
            pl.BlockSpec((D, D), lambda i: (0, 0)),
            pl.BlockSpec((1, D), lambda i: (0, 0)),
        ],
        out_specs=[pl.BlockSpec((nb, D), lambda i: (i, 0)),
                   pl.BlockSpec((nb, D), lambda i: (i, 0))],
        out_shape=[jax.ShapeDtypeStruct((N_NODES, D), jnp.float32),
                   jax.ShapeDtypeStruct((N_NODES, D), jnp.float32)],
    )(emb, wiT, bi2, wjT, bj2)


_FPB = 512


def _fp_stage(f_wide, wa, wb, e_pad):
    nrows = f_wide.shape[0]
    last_blk = (nrows - 1) // _FPB

    def body(f_ref, wa_ref, wb_ref, out_ref):
        i = pl.program_id(0)
        fb = f_ref[...]
        ma = jnp.dot(fb, wa_ref[...], preferred_element_type=jnp.float32)
        mb = jnp.dot(fb, wb_ref[...], preferred_element_type=jnp.float32)
        au = lax.bitcast_convert_type(ma, jnp.uint32) + 0x8000
        bu = lax.bitcast_convert_type(mb, jnp.uint32) + 0x8000
        out_ref[...] = (bu & jnp.uint32(0xFFFF0000)) | (au >> 16)

        @pl.when(i >= nrows // _FPB)
        def _mask_tail():
            row = lax.broadcasted_iota(jnp.int32, (_FPB, 8 * DH), 0) + i * _FPB
            out_ref[...] = jnp.where(row < nrows, out_ref[...], jnp.uint32(0))

    return pl.pallas_call(
        body,
        grid=(e_pad // 8 // _FPB,),
        in_specs=[
            pl.BlockSpec((_FPB, D), lambda i: (jnp.minimum(i, last_blk), 0)),
            pl.BlockSpec((D, 8 * DH), lambda i: (0, 0)),
            pl.BlockSpec((D, 8 * DH), lambda i: (0, 0)),
        ],
        out_specs=pl.BlockSpec((_FPB, 8 * DH), lambda i: (i, 0)),
        out_shape=jax.ShapeDtypeStruct((e_pad // 8, 8 * DH), jnp.uint32),
    )(f_wide, wa, wb)


def _make_edge_stage(e_pad):
    epw = e_pad // NW
    nchunk = epw // CH
    cnt0 = max(2, int(2 * nchunk * 0.63) // 2 * 2)
    cnt1 = 2 * nchunk - cnt0

    mesh = plsc.VectorSubcoreMesh(core_axis_name="c", subcore_axis_name="s")

    @functools.partial(
        pl.kernel,
        out_type=jax.ShapeDtypeStruct((NC * N_PAD, D), jnp.float32),
        mesh=mesh,
        scratch_types=[
            pltpu.VMEM((cnt0 * CH,), jnp.uint32),
            pltpu.VMEM((2, CH), jnp.int32),
            pltpu.VMEM((2, CH), jnp.int32),
            pltpu.VMEM((CH, D), jnp.float32),
            pltpu.VMEM((CH, D), jnp.float32),
            pltpu.VMEM((2, CH // 8, 8 * DH), jnp.uint32),
            pltpu.VMEM_SHARED((N_PAD, D), jnp.float32),
            pltpu.SemaphoreType.DMA,
            pltpu.SemaphoreType.DMA,
        ],
    )
    def edge_kernel(y_hbm, idx_hbm, fp_hbm, zeros_hbm, out_hbm,
                    pidx_v, idxj_r, idxi_r, rows0_v, rows1_v, fp_r,
                    acc_sh, sem0, sem1):
        c = lax.axis_index("c")
        s = lax.axis_index("s")
        row0 = s * RPT
        nch = jnp.where(c == 0, cnt0, cnt1)
        base0 = jnp.where(c == 0, s * cnt0, NS * cnt0 + s * cnt1) * CH

        pltpu.sync_copy(zeros_hbm, rows0_v)
        for off, n in _RPT_CHUNKS:
            pltpu.sync_copy(rows0_v.at[pl.ds(0, n)],
                            acc_sh.at[pl.ds(row0 + off, n)])

        pltpu.sync_copy(idx_hbm.at[pl.ds(base0, cnt0 * CH)], pidx_v)
        plsc.subcore_barrier()

        bufs = ((rows0_v, sem0), (rows1_v, sem1))

        def unpack_idx(kb, b):
            off = jnp.minimum(kb, nch - 1) * CH
            for g in range(CH // 16):
                u = pidx_v[pl.ds(off + g * 16, 16)]
                sl = pl.ds(g * 16, 16)
                idxj_r[b, sl] = (u >> 16).astype(jnp.int32)
                idxi_r[b, sl] = (u & 0xFFFF).astype(jnp.int32)

        def issue(kb, b):
            kc = jnp.minimum(kb, nch - 1)
            rows, sem = bufs[b]
            pltpu.async_copy(y_hbm.at[idxj_r.at[b]], rows, sem)
            pltpu.async_copy(
                fp_hbm.at[pl.ds(pl.multiple_of((base0 + kc * CH) // 8, 8),
                                CH // 8)],
                fp_r.at[b], sem)

        def drain(kb, b):
            kc = jnp.minimum(kb, nch - 1)
            rows, sem = bufs[b]
            pltpu.make_async_copy(y_hbm.at[idxj_r.at[b]], rows, sem).wait()
            pltpu.make_async_copy(
                fp_hbm.at[pl.ds(pl.multiple_of((base0 + kc * CH) // 8, 8),
                                CH // 8)],
                fp_r.at[b], sem).wait()

        unpack_idx(0, 0)
        issue(0, 0)
        unpack_idx(1, 1)
        issue(1, 1)

        himask = jnp.uint32(0xFFFF0000)

        def pair_body(i, carry):
            k = i * 2
            for b in range(2):
                kb = k + b
                rows, sem = bufs[b]
                drain(kb, b)

                def mul_body(q, c2):
                    fr = q >> 3
                    fc = (q & 7) * DH
                    ng = D // 32
                    ufs = [fp_r[b, fr, pl.ds(fc + g * 16, 16)]
                           for g in range(ng)]
                    ylos = [rows[q, pl.ds(g * 32, 16)] for g in range(ng)]
                    yhis = [rows[q, pl.ds(g * 32 + 16, 16)] for g in range(ng)]
                    plos = [y * lax.bitcast_convert_type(u << 16, jnp.float32)
                            for y, u in zip(ylos, ufs)]
                    phis = [y * lax.bitcast_convert_type(u & himask, jnp.float32)
                            for y, u in zip(yhis, ufs)]
                    for g in range(ng):
                        rows[q, pl.ds(g * 32, 16)] = plos[g]
                        rows[q, pl.ds(g * 32 + 16, 16)] = phis[g]
                    return c2

                lax.fori_loop(0, CH, mul_body, 0, unroll=4)
                pltpu.sync_copy(rows, acc_sh.at[idxi_r.at[b]], add=True)
                unpack_idx(kb + 2, b)
                issue(kb + 2, b)
            return carry

        lax.fori_loop(0, nch // 2, pair_body, 0)
        drain(nch, 0)
        drain(nch + 1, 1)
        plsc.subcore_barrier()

        out0 = c * N_PAD + row0
        for off, n in _RPT_CHUNKS:
            pltpu.sync_copy(acc_sh.at[pl.ds(row0 + off, n)],
                            rows0_v.at[pl.ds(0, n)])
            pltpu.sync_copy(rows0_v.at[pl.ds(0, n)],
                            out_hbm.at[pl.ds(out0 + off, n)])

    return edge_kernel


def _out_body(xi_ref, part_ref, emb_ref, w1T_ref, b1_ref, w2T_ref, b2_ref,
              wvT_ref, bv_ref, gate_ref, out_ref):
    m = xi_ref[...] + part_ref[0] + part_ref[1]
    for r in range(3):
        h = _softplus(
            jnp.dot(m, w1T_ref[r], preferred_element_type=jnp.float32)
            + b1_ref[r:r + 1, :])
        h = jnp.dot(h, w2T_ref[r], preferred_element_type=jnp.float32) \
            + b2_ref[r:r + 1, :]
        m = m + h
    v = jnp.dot(_softplus(m), wvT_ref[...],
                preferred_element_type=jnp.float32) + bv_ref[...]
    out_ref[...] = jax.nn.sigmoid(gate_ref[...]) * emb_ref[...] + v


def _out_stage(xi, parts3, emb, w1T, b1, w2T, b2, wvT, bv2, gate2):
    nb = 1000
    return pl.pallas_call(
        _out_body,
        grid=(N_NODES // nb,),
        in_specs=[
            pl.BlockSpec((nb, D), lambda i: (i, 0)),
            pl.BlockSpec((NC, nb, D), lambda i: (0, i, 0)),
            pl.BlockSpec((nb, D), lambda i: (i, 0)),
            pl.BlockSpec((3, D, D), lambda i: (0, 0, 0)),
            pl.BlockSpec((3, D), lambda i: (0, 0)),
            pl.BlockSpec((3, D, D), lambda i: (0, 0, 0)),
            pl.BlockSpec((3, D), lambda i: (0, 0)),
            pl.BlockSpec((D, D), lambda i: (0, 0)),
            pl.BlockSpec((1, D), lambda i: (0, 0)),
            pl.BlockSpec((1, D), lambda i: (0, 0)),
        ],
        out_specs=pl.BlockSpec((nb, D), lambda i: (i, 0)),
        out_shape=jax.ShapeDtypeStruct((N_NODES, D), jnp.float32),
    )(xi, parts3, emb, w1T, b1, w2T, b2, wvT, bv2, gate2)


def kernel(atomic_embedding, pair_indices, f_ij, d_ij, G, Wi, bi, Wj, bj,
           Wv, bv, res_W1, res_b1, res_W2, res_b2, gate):
    e = pair_indices.shape[1]
    npc = -(-e // (NW * CH))
    npc += npc % 2
    e_pad = NW * CH * npc
    pad = e_pad - e
    npc0 = max(2, int(2 * npc * 0.63) // 2 * 2)
    packed_idx = jnp.pad(
        (pair_indices[1].astype(jnp.uint32) << 16)
        | pair_indices[0].astype(jnp.uint32),
        (0, pad + (2 * npc0 - 2 * npc) * CH))

    gT = G.T
    eye8 = jnp.eye(8, dtype=jnp.float32)
    wa = jnp.kron(eye8, gT[:, _COLS_A]).astype(jnp.bfloat16)
    wb = jnp.kron(eye8, gT[:, _COLS_B]).astype(jnp.bfloat16)
    xi, y = _node_stage(atomic_embedding, Wi.T, bi.reshape(1, D),
                        Wj.T, bj.reshape(1, D))
    fpu = _fp_stage(f_ij.reshape(e // 8, 8 * 16).astype(jnp.bfloat16),
                    wa, wb, e_pad)

    zeros = jnp.zeros((CH, D), jnp.float32)
    parts = _make_edge_stage(e_pad)(y, packed_idx, fpu, zeros)
    parts3 = parts.reshape(NC, N_PAD, D)

    return _out_stage(xi, parts3, atomic_embedding,
                      res_W1.transpose(0, 2, 1), res_b1,
                      res_W2.transpose(0, 2, 1), res_b2,
                      Wv.T, bv.reshape(1, D), gate.reshape(1, D))


# speedup vs baseline: 1.8504x; 1.0126x over previous
import functools

import numpy as np
import jax
import jax.numpy as jnp
from jax import lax
from jax.experimental import pallas as pl
from jax.experimental.pallas import tpu as pltpu
from jax.experimental.pallas import tpu_sc as plsc

N_NODES = 10000
D = 128
DH = D // 2

NC = 2
NS = 16
NW = NC * NS
CH = 64
N_PAD = 10240
RPT = N_PAD // NS

_COLS_A = (np.arange(DH) // 16) * 32 + np.arange(DH) % 16
_COLS_B = _COLS_A + 16

_RPT_CHUNKS = [(t * CH, CH) for t in range(RPT // CH)]
if RPT % CH:
    _RPT_CHUNKS.append(((RPT // CH) * CH, RPT % CH))

_LOG2 = 0.6931471805599453


def _softplus(x):
    return jnp.logaddexp(x, 0.0)


def _pack_bf16_pairs(a, b):
    au = lax.bitcast_convert_type(a.astype(jnp.bfloat16), jnp.uint16)
    bu = lax.bitcast_convert_type(b.astype(jnp.bfloat16), jnp.uint16)
    return au.astype(jnp.uint32) | (bu.astype(jnp.uint32) << 16)


def _node_body(emb_ref, wiT_ref, bi_ref, wjT_ref, bj_ref, xi_ref, y_ref):
    x = _softplus(emb_ref[...]) - _LOG2
    xi_ref[...] = _softplus(
        jnp.dot(x, wiT_ref[...], preferred_element_type=jnp.float32) + bi_ref[...])
    y_ref[...] = _softplus(
        jnp.dot(x, wjT_ref[...], preferred_element_type=jnp.float32) + bj_ref[...])


def _node_stage(emb, wiT, bi2, wjT, bj2):
    nb = 1000
    return pl.pallas_call(
        _node_body,
        grid=(N_NODES // nb,),
        in_specs=[
            pl.BlockSpec((nb, D), lambda i: (i, 0)),
            pl.BlockSpec((D, D), lambda i: (0, 0)),
            pl.BlockSpec((1, D), lambda i: (0, 0)),
            pl.BlockSpec((D, D), lambda i: (0, 0)),
            pl.BlockSpec((1, D), lambda i: (0, 0)),
        ],
        out_specs=[pl.BlockSpec((nb, D), lambda i: (i, 0)),
                   pl.BlockSpec((nb, D), lambda i: (i, 0))],
        out_shape=[jax.ShapeDtypeStruct((N_NODES, D), jnp.float32),
                   jax.ShapeDtypeStruct((N_NODES, D), jnp.float32)],
    )(emb, wiT, bi2, wjT, bj2)


_FPB = 512


def _fp_stage(f_wide, wa, wb, e_pad):
    nrows = f_wide.shape[0]
    last_blk = (nrows - 1) // _FPB

    def body(f_ref, wa_ref, wb_ref, out_ref):
        i = pl.program_id(0)
        fb = f_ref[...]
        ma = jnp.dot(fb, wa_ref[...], preferred_element_type=jnp.float32)
        mb = jnp.dot(fb, wb_ref[...], preferred_element_type=jnp.float32)
        au = lax.bitcast_convert_type(ma, jnp.uint32) + 0x8000
        bu = lax.bitcast_convert_type(mb, jnp.uint32) + 0x8000
        out_ref[...] = (bu & jnp.uint32(0xFFFF0000)) | (au >> 16)

        @pl.when(i >= nrows // _FPB)
        def _mask_tail():
            row = lax.broadcasted_iota(jnp.int32, (_FPB, 8 * DH), 0) + i * _FPB
            out_ref[...] = jnp.where(row < nrows, out_ref[...], jnp.uint32(0))

    return pl.pallas_call(
        body,
        grid=(e_pad // 8 // _FPB,),
        in_specs=[
            pl.BlockSpec((_FPB, D), lambda i: (jnp.minimum(i, last_blk), 0)),
            pl.BlockSpec((D, 8 * DH), lambda i: (0, 0)),
            pl.BlockSpec((D, 8 * DH), lambda i: (0, 0)),
        ],
        out_specs=pl.BlockSpec((_FPB, 8 * DH), lambda i: (i, 0)),
        out_shape=jax.ShapeDtypeStruct((e_pad // 8, 8 * DH), jnp.uint32),
    )(f_wide, wa, wb)


def _make_edge_stage(e_pad):
    epw = e_pad // NW
    nchunk = epw // CH
    cnt0 = max(2, int(2 * nchunk * 0.66) // 2 * 2)
    cnt1 = 2 * nchunk - cnt0

    mesh = plsc.VectorSubcoreMesh(core_axis_name="c", subcore_axis_name="s")

    @functools.partial(
        pl.kernel,
        out_type=jax.ShapeDtypeStruct((NC * N_PAD, D), jnp.float32),
        mesh=mesh,
        scratch_types=[
            pltpu.VMEM((cnt0 * CH,), jnp.uint32),
            pltpu.VMEM((2, CH), jnp.int32),
            pltpu.VMEM((2, CH), jnp.int32),
            pltpu.VMEM((CH, D), jnp.float32),
            pltpu.VMEM((CH, D), jnp.float32),
            pltpu.VMEM((2, CH // 8, 8 * DH), jnp.uint32),
            pltpu.VMEM_SHARED((N_PAD, D), jnp.float32),
            pltpu.SemaphoreType.DMA,
            pltpu.SemaphoreType.DMA,
        ],
    )
    def edge_kernel(y_hbm, idx_hbm, fp_hbm, zeros_hbm, out_hbm,
                    pidx_v, idxj_r, idxi_r, rows0_v, rows1_v, fp_r,
                    acc_sh, sem0, sem1):
        c = lax.axis_index("c")
        s = lax.axis_index("s")
        row0 = s * RPT
        nch = jnp.where(c == 0, cnt0, cnt1)
        base0 = jnp.where(c == 0, s * cnt0, NS * cnt0 + s * cnt1) * CH

        pltpu.sync_copy(zeros_hbm, rows0_v)
        for off, n in _RPT_CHUNKS:
            pltpu.sync_copy(rows0_v.at[pl.ds(0, n)],
                            acc_sh.at[pl.ds(row0 + off, n)])

        pltpu.sync_copy(idx_hbm.at[pl.ds(base0, cnt0 * CH)], pidx_v)
        plsc.subcore_barrier()

        bufs = ((rows0_v, sem0), (rows1_v, sem1))

        def unpack_idx(kb, b):
            off = jnp.minimum(kb, nch - 1) * CH
            for g in range(CH // 16):
                u = pidx_v[pl.ds(off + g * 16, 16)]
                sl = pl.ds(g * 16, 16)
                idxj_r[b, sl] = (u >> 16).astype(jnp.int32)
                idxi_r[b, sl] = (u & 0xFFFF).astype(jnp.int32)

        def issue(kb, b):
            kc = jnp.minimum(kb, nch - 1)
            rows, sem = bufs[b]
            pltpu.async_copy(y_hbm.at[idxj_r.at[b]], rows, sem)
            pltpu.async_copy(
                fp_hbm.at[pl.ds(pl.multiple_of((base0 + kc * CH) // 8, 8),
                                CH // 8)],
                fp_r.at[b], sem)

        def drain(kb, b):
            kc = jnp.minimum(kb, nch - 1)
            rows, sem = bufs[b]
            pltpu.make_async_copy(y_hbm.at[idxj_r.at[b]], rows, sem).wait()
            pltpu.make_async_copy(
                fp_hbm.at[pl.ds(pl.multiple_of((base0 + kc * CH) // 8, 8),
                                CH // 8)],
                fp_r.at[b], sem).wait()

        unpack_idx(0, 0)
        issue(0, 0)
        unpack_idx(1, 1)
        issue(1, 1)

        himask = jnp.uint32(0xFFFF0000)

        def pair_body(i, carry):
            k = i * 2
            for b in range(2):
                kb = k + b
                rows, sem = bufs[b]
                drain(kb, b)

                def mul_body(q, c2):
                    fr = q >> 3
                    fc = (q & 7) * DH
                    ng = D // 32
                    ufs = [fp_r[b, fr, pl.ds(fc + g * 16, 16)]
                           for g in range(ng)]
                    ylos = [rows[q, pl.ds(g * 32, 16)] for g in range(ng)]
                    yhis = [rows[q, pl.ds(g * 32 + 16, 16)] for g in range(ng)]
                    plos = [y * lax.bitcast_convert_type(u << 16, jnp.float32)
                            for y, u in zip(ylos, ufs)]
                    phis = [y * lax.bitcast_convert_type(u & himask, jnp.float32)
                            for y, u in zip(yhis, ufs)]
                    for g in range(ng):
                        rows[q, pl.ds(g * 32, 16)] = plos[g]
                        rows[q, pl.ds(g * 32 + 16, 16)] = phis[g]
                    return c2

                lax.fori_loop(0, CH, mul_body, 0, unroll=4)
                pltpu.sync_copy(rows, acc_sh.at[idxi_r.at[b]], add=True)
                unpack_idx(kb + 2, b)
                issue(kb + 2, b)
            return carry

        lax.fori_loop(0, nch // 2, pair_body, 0)
        drain(nch, 0)
        drain(nch + 1, 1)
        plsc.subcore_barrier()

        out0 = c * N_PAD + row0
        for off, n in _RPT_CHUNKS:
            pltpu.sync_copy(acc_sh.at[pl.ds(row0 + off, n)],
                            rows0_v.at[pl.ds(0, n)])
            pltpu.sync_copy(rows0_v.at[pl.ds(0, n)],
                            out_hbm.at[pl.ds(out0 + off, n)])

    return edge_kernel


def _out_body(xi_ref, part_ref, emb_ref, w1T_ref, b1_ref, w2T_ref, b2_ref,
              wvT_ref, bv_ref, gate_ref, out_ref):
    m = xi_ref[...] + part_ref[0] + part_ref[1]
    for r in range(3):
        h = _softplus(
            jnp.dot(m, w1T_ref[r], preferred_element_type=jnp.float32)
            + b1_ref[r:r + 1, :])
        h = jnp.dot(h, w2T_ref[r], preferred_element_type=jnp.float32) \
            + b2_ref[r:r + 1, :]
        m = m + h
    v = jnp.dot(_softplus(m), wvT_ref[...],
                preferred_element_type=jnp.float32) + bv_ref[...]
    out_ref[...] = jax.nn.sigmoid(gate_ref[...]) * emb_ref[...] + v


def _out_stage(xi, parts3, emb, w1T, b1, w2T, b2, wvT, bv2, gate2):
    nb = 1000
    return pl.pallas_call(
        _out_body,
        grid=(N_NODES // nb,),
        in_specs=[
            pl.BlockSpec((nb, D), lambda i: (i, 0)),
            pl.BlockSpec((NC, nb, D), lambda i: (0, i, 0)),
            pl.BlockSpec((nb, D), lambda i: (i, 0)),
            pl.BlockSpec((3, D, D), lambda i: (0, 0, 0)),
            pl.BlockSpec((3, D), lambda i: (0, 0)),
            pl.BlockSpec((3, D, D), lambda i: (0, 0, 0)),
            pl.BlockSpec((3, D), lambda i: (0, 0)),
            pl.BlockSpec((D, D), lambda i: (0, 0)),
            pl.BlockSpec((1, D), lambda i: (0, 0)),
            pl.BlockSpec((1, D), lambda i: (0, 0)),
        ],
        out_specs=pl.BlockSpec((nb, D), lambda i: (i, 0)),
        out_shape=jax.ShapeDtypeStruct((N_NODES, D), jnp.float32),
    )(xi, parts3, emb, w1T, b1, w2T, b2, wvT, bv2, gate2)


def kernel(atomic_embedding, pair_indices, f_ij, d_ij, G, Wi, bi, Wj, bj,
           Wv, bv, res_W1, res_b1, res_W2, res_b2, gate):
    e = pair_indices.shape[1]
    npc = -(-e // (NW * CH))
    npc += npc % 2
    e_pad = NW * CH * npc
    pad = e_pad - e
    npc0 = max(2, int(2 * npc * 0.66) // 2 * 2)
    packed_idx = jnp.pad(
        (pair_indices[1].astype(jnp.uint32) << 16)
        | pair_indices[0].astype(jnp.uint32),
        (0, pad + (2 * npc0 - 2 * npc) * CH))

    gT = G.T
    eye8 = jnp.eye(8, dtype=jnp.float32)
    wa = jnp.kron(eye8, gT[:, _COLS_A]).astype(jnp.bfloat16)
    wb = jnp.kron(eye8, gT[:, _COLS_B]).astype(jnp.bfloat16)
    xi, y = _node_stage(atomic_embedding, Wi.T, bi.reshape(1, D),
                        Wj.T, bj.reshape(1, D))
    fpu = _fp_stage(f_ij.reshape(e // 8, 8 * 16).astype(jnp.bfloat16),
                    wa, wb, e_pad)

    zeros = jnp.zeros((CH, D), jnp.float32)
    parts = _make_edge_stage(e_pad)(y, packed_idx, fpu, zeros)
    parts3 = parts.reshape(NC, N_PAD, D)

    return _out_stage(xi, parts3, atomic_embedding,
                      res_W1.transpose(0, 2, 1), res_b1,
                      res_W2.transpose(0, 2, 1), res_b2,
                      Wv.T, bv.reshape(1, D), gate.reshape(1, D))


# speedup vs baseline: 1.8515x; 1.0005x over previous
import functools

import numpy as np
import jax
import jax.numpy as jnp
from jax import lax
from jax.experimental import pallas as pl
from jax.experimental.pallas import tpu as pltpu
from jax.experimental.pallas import tpu_sc as plsc

N_NODES = 10000
D = 128
DH = D // 2

NC = 2
NS = 16
NW = NC * NS
CH = 64
N_PAD = 10240
RPT = N_PAD // NS

_COLS_A = (np.arange(DH) // 16) * 32 + np.arange(DH) % 16
_COLS_B = _COLS_A + 16

_RPT_CHUNKS = [(t * CH, CH) for t in range(RPT // CH)]
if RPT % CH:
    _RPT_CHUNKS.append(((RPT // CH) * CH, RPT % CH))

_LOG2 = 0.6931471805599453


def _softplus(x):
    return jnp.logaddexp(x, 0.0)


def _node_body(emb_ref, wiT_ref, bi_ref, wjT_ref, bj_ref, xi_ref, y_ref):
    x = _softplus(emb_ref[...]) - _LOG2
    xi_ref[...] = _softplus(
        jnp.dot(x, wiT_ref[...], preferred_element_type=jnp.float32) + bi_ref[...])
    y_ref[...] = _softplus(
        jnp.dot(x, wjT_ref[...], preferred_element_type=jnp.float32) + bj_ref[...])


def _node_stage(emb, wiT, bi2, wjT, bj2):
    nb = 1000
    return pl.pallas_call(
        _node_body,
        grid=(N_NODES // nb,),
        in_specs=[
            pl.BlockSpec((nb, D), lambda i: (i, 0)),
            pl.BlockSpec((D, D), lambda i: (0, 0)),
            pl.BlockSpec((1, D), lambda i: (0, 0)),
            pl.BlockSpec((D, D), lambda i: (0, 0)),
            pl.BlockSpec((1, D), lambda i: (0, 0)),
        ],
        out_specs=[pl.BlockSpec((nb, D), lambda i: (i, 0)),
                   pl.BlockSpec((nb, D), lambda i: (i, 0))],
        out_shape=[jax.ShapeDtypeStruct((N_NODES, D), jnp.float32),
                   jax.ShapeDtypeStruct((N_NODES, D), jnp.float32)],
    )(emb, wiT, bi2, wjT, bj2)


_FPB = 512


def _fp_stage(f_wide, wa, wb, e_pad):
    nrows = f_wide.shape[0]
    last_blk = (nrows - 1) // _FPB

    def body(f_ref, wa_ref, wb_ref, out_ref):
        i = pl.program_id(0)
        fb = f_ref[...]
        ma = jnp.dot(fb, wa_ref[...], preferred_element_type=jnp.float32)
        mb = jnp.dot(fb, wb_ref[...], preferred_element_type=jnp.float32)
        au = lax.bitcast_convert_type(ma, jnp.uint32) + 0x8000
        bu = lax.bitcast_convert_type(mb, jnp.uint32) + 0x8000
        out_ref[...] = (bu & jnp.uint32(0xFFFF0000)) | (au >> 16)

        @pl.when(i >= nrows // _FPB)
        def _mask_tail():
            row = lax.broadcasted_iota(jnp.int32, (_FPB, 8 * DH), 0) + i * _FPB
            out_ref[...] = jnp.where(row < nrows, out_ref[...], jnp.uint32(0))

    return pl.pallas_call(
        body,
        grid=(e_pad // 8 // _FPB,),
        in_specs=[
            pl.BlockSpec((_FPB, D), lambda i: (jnp.minimum(i, last_blk), 0)),
            pl.BlockSpec((D, 8 * DH), lambda i: (0, 0)),
            pl.BlockSpec((D, 8 * DH), lambda i: (0, 0)),
        ],
        out_specs=pl.BlockSpec((_FPB, 8 * DH), lambda i: (i, 0)),
        out_shape=jax.ShapeDtypeStruct((e_pad // 8, 8 * DH), jnp.uint32),
    )(f_wide, wa, wb)


def _make_edge_stage(e_pad):
    epw = e_pad // NW
    nchunk = epw // CH
    cnt0 = max(2, int(2 * nchunk * 0.66) // 2 * 2)
    cnt1 = 2 * nchunk - cnt0

    mesh = plsc.VectorSubcoreMesh(core_axis_name="c", subcore_axis_name="s")

    @functools.partial(
        pl.kernel,
        out_type=jax.ShapeDtypeStruct((NC * N_PAD, D), jnp.float32),
        mesh=mesh,
        scratch_types=[
            pltpu.VMEM((cnt0 * CH,), jnp.uint32),
            pltpu.VMEM((2, CH), jnp.int32),
            pltpu.VMEM((2, CH), jnp.int32),
            pltpu.VMEM((CH, D), jnp.float32),
            pltpu.VMEM((CH, D), jnp.float32),
            pltpu.VMEM((2, CH // 8, 8 * DH), jnp.uint32),
            pltpu.VMEM_SHARED((N_PAD, D), jnp.float32),
            pltpu.SemaphoreType.DMA,
            pltpu.SemaphoreType.DMA,
        ],
    )
    def edge_kernel(y_hbm, idx_hbm, fp_hbm, zeros_hbm, out_hbm,
                    pidx_v, idxj_r, idxi_r, rows0_v, rows1_v, fp_r,
                    acc_sh, sem0, sem1):
        c = lax.axis_index("c")
        s = lax.axis_index("s")
        row0 = s * RPT
        nch = jnp.where(c == 0, cnt0, cnt1)
        base0 = jnp.where(c == 0, s * cnt0, NS * cnt0 + s * cnt1) * CH

        pltpu.sync_copy(zeros_hbm, rows0_v)
        for off, n in _RPT_CHUNKS:
            pltpu.sync_copy(rows0_v.at[pl.ds(0, n)],
                            acc_sh.at[pl.ds(row0 + off, n)])

        pltpu.sync_copy(idx_hbm.at[pl.ds(base0, cnt0 * CH)], pidx_v)
        plsc.subcore_barrier()

        bufs = ((rows0_v, sem0), (rows1_v, sem1))

        def unpack_idx(kb, b):
            off = jnp.minimum(kb, nch - 1) * CH
            for g in range(CH // 16):
                u = pidx_v[pl.ds(off + g * 16, 16)]
                sl = pl.ds(g * 16, 16)
                idxj_r[b, sl] = (u >> 16).astype(jnp.int32)
                idxi_r[b, sl] = (u & 0xFFFF).astype(jnp.int32)

        def issue(kb, b):
            kc = jnp.minimum(kb, nch - 1)
            rows, sem = bufs[b]
            pltpu.async_copy(y_hbm.at[idxj_r.at[b]], rows, sem)
            pltpu.async_copy(
                fp_hbm.at[pl.ds(pl.multiple_of((base0 + kc * CH) // 8, 8),
                                CH // 8)],
                fp_r.at[b], sem)

        def drain(kb, b):
            kc = jnp.minimum(kb, nch - 1)
            rows, sem = bufs[b]
            pltpu.make_async_copy(y_hbm.at[idxj_r.at[b]], rows, sem).wait()
            pltpu.make_async_copy(
                fp_hbm.at[pl.ds(pl.multiple_of((base0 + kc * CH) // 8, 8),
                                CH // 8)],
                fp_r.at[b], sem).wait()

        unpack_idx(0, 0)
        issue(0, 0)
        unpack_idx(1, 1)
        issue(1, 1)

        himask = jnp.uint32(0xFFFF0000)

        def pair_body(i, carry):
            k = i * 2
            for b in range(2):
                kb = k + b
                rows, sem = bufs[b]
                drain(kb, b)

                def mul_body(q, c2):
                    fr = q >> 3
                    fc = (q & 7) * DH
                    ng = D // 32
                    ufs = [fp_r[b, fr, pl.ds(fc + g * 16, 16)]
                           for g in range(ng)]
                    ylos = [rows[q, pl.ds(g * 32, 16)] for g in range(ng)]
                    yhis = [rows[q, pl.ds(g * 32 + 16, 16)] for g in range(ng)]
                    plos = [y * lax.bitcast_convert_type(u << 16, jnp.float32)
                            for y, u in zip(ylos, ufs)]
                    phis = [y * lax.bitcast_convert_type(u & himask, jnp.float32)
                            for y, u in zip(yhis, ufs)]
                    for g in range(ng):
                        rows[q, pl.ds(g * 32, 16)] = plos[g]
                        rows[q, pl.ds(g * 32 + 16, 16)] = phis[g]
                    return c2

                lax.fori_loop(0, CH, mul_body, 0, unroll=4)
                pltpu.sync_copy(rows, acc_sh.at[idxi_r.at[b]], add=True)
                unpack_idx(kb + 2, b)
                issue(kb + 2, b)
            return carry

        lax.fori_loop(0, nch // 2, pair_body, 0)
        drain(nch, 0)
        drain(nch + 1, 1)
        plsc.subcore_barrier()

        out0 = c * N_PAD + row0
        for off, n in _RPT_CHUNKS:
            pltpu.sync_copy(acc_sh.at[pl.ds(row0 + off, n)],
                            rows0_v.at[pl.ds(0, n)])
            pltpu.sync_copy(rows0_v.at[pl.ds(0, n)],
                            out_hbm.at[pl.ds(out0 + off, n)])

    return edge_kernel


def _out_body(xi_ref, part_ref, emb_ref, w1T_ref, b1_ref, w2T_ref, b2_ref,
              wvT_ref, bv_ref, gate_ref, out_ref):
    m = xi_ref[...] + part_ref[0] + part_ref[1]
    for r in range(3):
        h = _softplus(
            jnp.dot(m, w1T_ref[r], preferred_element_type=jnp.float32)
            + b1_ref[r:r + 1, :])
        h = jnp.dot(h, w2T_ref[r], preferred_element_type=jnp.float32) \
            + b2_ref[r:r + 1, :]
        m = m + h
    v = jnp.dot(_softplus(m), wvT_ref[...],
                preferred_element_type=jnp.float32) + bv_ref[...]
    out_ref[...] = jax.nn.sigmoid(gate_ref[...]) * emb_ref[...] + v


def _out_stage(xi, parts3, emb, w1T, b1, w2T, b2, wvT, bv2, gate2):
    nb = 1000
    return pl.pallas_call(
        _out_body,
        grid=(N_NODES // nb,),
        in_specs=[
            pl.BlockSpec((nb, D), lambda i: (i, 0)),
            pl.BlockSpec((NC, nb, D), lambda i: (0, i, 0)),
            pl.BlockSpec((nb, D), lambda i: (i, 0)),
            pl.BlockSpec((3, D, D), lambda i: (0, 0, 0)),
            pl.BlockSpec((3, D), lambda i: (0, 0)),
            pl.BlockSpec((3, D, D), lambda i: (0, 0, 0)),
            pl.BlockSpec((3, D), lambda i: (0, 0)),
            pl.BlockSpec((D, D), lambda i: (0, 0)),
            pl.BlockSpec((1, D), lambda i: (0, 0)),
            pl.BlockSpec((1, D), lambda i: (0, 0)),
        ],
        out_specs=pl.BlockSpec((nb, D), lambda i: (i, 0)),
        out_shape=jax.ShapeDtypeStruct((N_NODES, D), jnp.float32),
    )(xi, parts3, emb, w1T, b1, w2T, b2, wvT, bv2, gate2)


def kernel(atomic_embedding, pair_indices, f_ij, d_ij, G, Wi, bi, Wj, bj,
           Wv, bv, res_W1, res_b1, res_W2, res_b2, gate):
    e = pair_indices.shape[1]
    npc = -(-e // (NW * CH))
    npc += npc % 2
    e_pad = NW * CH * npc
    pad = e_pad - e
    npc0 = max(2, int(2 * npc * 0.66) // 2 * 2)
    packed_idx = jnp.pad(
        (pair_indices[1].astype(jnp.uint32) << 16)
        | pair_indices[0].astype(jnp.uint32),
        (0, pad + (2 * npc0 - 2 * npc) * CH))

    gT = G.T
    eye8 = jnp.eye(8, dtype=jnp.float32)
    wa = jnp.kron(eye8, gT[:, _COLS_A]).astype(jnp.bfloat16)
    wb = jnp.kron(eye8, gT[:, _COLS_B]).astype(jnp.bfloat16)
    xi, y = _node_stage(atomic_embedding, Wi.T, bi.reshape(1, D),
                        Wj.T, bj.reshape(1, D))
    fpu = _fp_stage(f_ij.reshape(e // 8, 8 * 16).astype(jnp.bfloat16),
                    wa, wb, e_pad)

    zeros = jnp.zeros((CH, D), jnp.float32)
    parts = _make_edge_stage(e_pad)(y, packed_idx, fpu, zeros)
    parts3 = parts.reshape(NC, N_PAD, D)

    return _out_stage(xi, parts3, atomic_embedding,
                      res_W1.transpose(0, 2, 1), res_b1,
                      res_W2.transpose(0, 2, 1), res_b2,
                      Wv.T, bv.reshape(1, D), gate.reshape(1, D))
